# Initial kernel scaffold; baseline (speedup 1.0000x reference)
import jax
import jax.numpy as jnp
from jax.experimental import pallas as pl


def kernel(conformations, native_state, W1, b1, W2, b2, W3, b3, Ws1, bs1, Ws2, bs2):
    raise NotImplementedError("write your pallas kernel here")


# speedup vs baseline: 13.0944x; 13.0944x over previous
import functools

import jax
import jax.numpy as jnp
from jax.experimental import pallas as pl
from jax.experimental.pallas import tpu as pltpu

_INF = float("inf")
_NEG_INF = float("-inf")


def _prologue_body(conf_ref, confbf_ref, native_ref, W1_ref, b1_ref,
                   W2_ref, b2_ref, W3_ref, b3_ref, Ws1_ref, bs1_ref,
                   Ws2_ref, bs2_ref,
                   e_ref, sid_ref, nd_ref, met_ref):
    x = conf_ref[0]
    xb = confbf_ref[0]
    n = x.shape[0]
    bf = jnp.bfloat16
    f32 = jnp.float32

    h1 = jnp.maximum(
        jnp.dot(xb, W1_ref[...], preferred_element_type=f32)
        + b1_ref[...], 0.0)
    h2 = jnp.maximum(
        jnp.dot(h1.astype(bf), W2_ref[...], preferred_element_type=f32)
        + b2_ref[...], 0.0)
    e8 = jnp.dot(h2.astype(bf), W3_ref[...], preferred_element_type=f32)
    e = e8[:, 0:1] + b3_ref[...]

    g = jnp.maximum(
        jnp.dot(xb, Ws1_ref[...], preferred_element_type=f32)
        + bs1_ref[...], 0.0)
    logits = jnp.dot(g.astype(bf), Ws2_ref[...], preferred_element_type=f32) \
        + bs2_ref[...]
    lmax = jnp.max(logits, axis=1, keepdims=True)
    cio = jax.lax.broadcasted_iota(jnp.int32, logits.shape, 1)
    sid = jnp.min(jnp.where(logits == lmax, cio, logits.shape[1]),
                  axis=1, keepdims=True)

    diff = x - native_ref[0]
    nd = jnp.sqrt(jnp.sum(diff * diff, axis=1, keepdims=True) + 1e-12)

    e_ref[0] = e
    sid_ref[0] = sid
    nd_ref[0] = nd

    nf = jnp.float32(n)
    mean_e = jnp.sum(e) / nf
    ce = e - mean_e
    se2 = jnp.sum(ce * ce)
    rugged = jnp.sqrt(se2 / (nf - 1.0))
    max_e = jnp.max(e)

    mean_nd = jnp.sum(nd) / nf
    cnd = nd - mean_nd
    snd2 = jnp.sum(cnd * cnd)
    corr = jnp.sum(cnd * ce) / jnp.sqrt(se2 * snd2)

    rio = jax.lax.broadcasted_iota(jnp.int32, nd.shape, 0)
    nd_min = jnp.min(nd)
    amin = jnp.min(jnp.where(nd == nd_min, rio, n))
    native_e = jnp.min(jnp.where(rio == amin, e, _INF))
    funnel = max_e - native_e

    lane = jax.lax.broadcasted_iota(jnp.int32, (1, 128), 1)
    met = (jnp.where(lane == 0, funnel, 0.0)
           + jnp.where(lane == 1, rugged, 0.0)
           + jnp.where(lane == 2, corr, 0.0)
           + jnp.where(lane == 3, native_e, 0.0))
    met_ref[0] = met


def _pair_body(xrow_ref, xT_ref, e_ref, sq_ref, nd_ref,
               basin_ref, depth_ref, width_ref, frus_ref, *, R, K):
    nb = pl.program_id(1)
    r0 = nb * R

    xb = xrow_ref[0]
    xT = xT_ref[0]
    e_r = e_ref[0]
    sq_r = sq_ref[0]
    nd_r = nd_ref[0]
    n = e_r.shape[1]

    dots = jnp.dot(xb, xT, preferred_element_type=jnp.float32)
    sq_b = jnp.reshape(sq_ref[0, 0, pl.ds(r0, R)], (R, 1))
    d2 = sq_b + sq_r - 2.0 * dots
    d = jnp.sqrt(jnp.maximum(d2, 0.0) + 1e-12)

    colio = jax.lax.broadcasted_iota(jnp.int32, (R, n), 1)

    dm = d
    neigh_min = jnp.full((R, 1), _INF, dtype=jnp.float32)
    m = None
    for k in range(K):
        m = jnp.min(dm, axis=1, keepdims=True)
        idx = jnp.min(jnp.where(dm == m, colio, n), axis=1, keepdims=True)
        sel = colio == idx
        if k >= 1:
            e_at = jnp.min(jnp.where(sel, e_r, _INF), axis=1, keepdims=True)
            neigh_min = jnp.minimum(neigh_min, e_at)
        dm = jnp.where(sel, _INF, dm)
    thresh = m

    e_b = jnp.reshape(e_ref[0, 0, pl.ds(r0, R)], (R, 1))
    is_basin = e_b <= neigh_min

    mask = d < thresh
    depth = jnp.max(jnp.where(mask, e_r, _NEG_INF), axis=1, keepdims=True) - e_b
    cnt = jnp.sum(mask.astype(jnp.float32), axis=1, keepdims=True)
    wsum = jnp.sum(jnp.where(mask, d, 0.0), axis=1, keepdims=True)
    width = wsum / jnp.maximum(cnt, 1.0)

    basin_ref[0, 0] = jnp.reshape(is_basin.astype(jnp.int32), (R,))
    depth_ref[0, 0] = jnp.reshape(jnp.where(is_basin, depth, 0.0), (R,))
    width_ref[0, 0] = jnp.reshape(jnp.where(is_basin, width, 0.0), (R,))

    nd_b = jnp.reshape(nd_ref[0, 0, pl.ds(r0, R)], (R, 1))
    gidx = r0 + jax.lax.broadcasted_iota(jnp.int32, (R, 1), 0)
    gt = (nd_r > nd_b) | ((nd_r == nd_b) & (colio > gidx))
    m2 = jnp.min(jnp.where(gt, nd_r, _INF), axis=1, keepdims=True)
    idx2 = jnp.min(jnp.where(gt & (nd_r == m2), colio, n),
                   axis=1, keepdims=True)
    e_succ = jnp.min(jnp.where(colio == idx2, e_r, _INF),
                     axis=1, keepdims=True)
    contrib = ((e_succ > e_b) & (idx2 < n)).astype(jnp.float32)
    part = jnp.sum(contrib)

    @pl.when(nb == 0)
    def _():
        frus_ref[0, 0] = jnp.zeros((128,), dtype=jnp.float32)

    frus_ref[0, 0] = frus_ref[0, 0] + part


def kernel(conformations, native_state, W1, b1, W2, b2, W3, b3,
           Ws1, bs1, Ws2, bs2):
    B, N, D = conformations.shape
    H = W1.shape[1]
    H2 = W2.shape[1]
    K = 10
    R = 256
    NB = N // R

    f32 = jnp.float32
    bf = jnp.bfloat16
    conf = conformations.astype(f32)
    conf_bf = conf.astype(bf)
    confT_bf = jnp.transpose(conf_bf, (0, 2, 1))

    sq = jnp.sum(conf * conf, axis=-1)
    sq_row = jnp.reshape(sq, (B, 1, N))

    C = Ws2.shape[1]
    Ws2p = jnp.zeros((H, 8), f32).at[:, :C].set(Ws2).astype(bf)
    bs2p = jnp.full((1, 8), -1e30, f32).at[0, :C].set(bs2)
    W3p = jnp.zeros((H2, 8), f32).at[:, 0:1].set(W3).astype(bf)

    b1r = jnp.reshape(b1, (1, H))
    b2r = jnp.reshape(b2, (1, H2))
    b3r = jnp.reshape(b3, (1, 1))
    bs1r = jnp.reshape(bs1, (1, H))
    nat = jnp.reshape(native_state, (B, 1, D))

    W1b = W1.astype(bf)
    W2b = W2.astype(bf)
    Ws1b = Ws1.astype(bf)

    wspec = lambda *shape: pl.BlockSpec(shape, lambda b: (0,) * len(shape))

    e_col, sid_col, nd_col, met = pl.pallas_call(
        _prologue_body,
        grid=(B,),
        in_specs=[
            pl.BlockSpec((1, N, D), lambda b: (b, 0, 0)),
            pl.BlockSpec((1, N, D), lambda b: (b, 0, 0)),
            pl.BlockSpec((1, 1, D), lambda b: (b, 0, 0)),
            wspec(D, H), wspec(1, H),
            wspec(H, H2), wspec(1, H2),
            wspec(H2, 8), wspec(1, 1),
            wspec(D, H), wspec(1, H),
            wspec(H, 8), wspec(1, 8),
        ],
        out_specs=[
            pl.BlockSpec((1, N, 1), lambda b: (b, 0, 0)),
            pl.BlockSpec((1, N, 1), lambda b: (b, 0, 0)),
            pl.BlockSpec((1, N, 1), lambda b: (b, 0, 0)),
            pl.BlockSpec((1, 1, 128), lambda b: (b, 0, 0)),
        ],
        out_shape=[
            jax.ShapeDtypeStruct((B, N, 1), f32),
            jax.ShapeDtypeStruct((B, N, 1), jnp.int32),
            jax.ShapeDtypeStruct((B, N, 1), f32),
            jax.ShapeDtypeStruct((B, 1, 128), f32),
        ],
    )(conf, conf_bf, nat, W1b, b1r, W2b, b2r, W3p, b3r, Ws1b, bs1r,
      Ws2p, bs2p)

    e_row = jnp.reshape(e_col, (B, 1, N))
    nd_row = jnp.reshape(nd_col, (B, 1, N))

    basin, depth, width, frus = pl.pallas_call(
        functools.partial(_pair_body, R=R, K=K),
        grid=(B, NB),
        in_specs=[
            pl.BlockSpec((1, R, D), lambda b, nb: (b, nb, 0)),
            pl.BlockSpec((1, D, N), lambda b, nb: (b, 0, 0)),
            pl.BlockSpec((1, 1, N), lambda b, nb: (b, 0, 0)),
            pl.BlockSpec((1, 1, N), lambda b, nb: (b, 0, 0)),
            pl.BlockSpec((1, 1, N), lambda b, nb: (b, 0, 0)),
        ],
        out_specs=[
            pl.BlockSpec((1, 1, R), lambda b, nb: (b * NB + nb, 0, 0)),
            pl.BlockSpec((1, 1, R), lambda b, nb: (b * NB + nb, 0, 0)),
            pl.BlockSpec((1, 1, R), lambda b, nb: (b * NB + nb, 0, 0)),
            pl.BlockSpec((1, 1, 128), lambda b, nb: (b, 0, 0)),
        ],
        out_shape=[
            jax.ShapeDtypeStruct((B * NB, 1, R), jnp.int32),
            jax.ShapeDtypeStruct((B * NB, 1, R), f32),
            jax.ShapeDtypeStruct((B * NB, 1, R), f32),
            jax.ShapeDtypeStruct((B, 1, 128), f32),
        ],
    )(conf_bf, confT_bf, e_row, sq_row, nd_row)

    e_out = jnp.reshape(e_col, (B, N))
    is_basin = jnp.reshape(basin, (B, N)) != 0
    depth_out = jnp.reshape(depth, (B, N))
    width_out = jnp.reshape(width, (B, N))
    sid_out = jnp.reshape(sid_col, (B, N))

    frustration = frus[:, 0, 0] / jnp.float32(N - 1)
    metrics = jnp.stack(
        [met[:, 0, 0], met[:, 0, 1], met[:, 0, 2], frustration,
         met[:, 0, 3]], axis=1)

    return (e_out, is_basin, depth_out, width_out, sid_out, metrics)


# speedup vs baseline: 14.5196x; 1.1088x over previous
import functools

import jax
import jax.numpy as jnp
from jax.experimental import pallas as pl
from jax.experimental.pallas import tpu as pltpu

_INF = float("inf")
_NEG_INF = float("-inf")


def _prologue_body(conf_ref, confbf_ref, native_ref, W1_ref, b1_ref,
                   W2_ref, b2_ref, W3_ref, b3_ref, Ws1_ref, bs1_ref,
                   Ws2_ref, bs2_ref,
                   e_ref, sid_ref, nd_ref, met_ref):
    x = conf_ref[0]
    xb = confbf_ref[0]
    n = x.shape[0]
    bf = jnp.bfloat16
    f32 = jnp.float32

    h1 = jnp.maximum(
        jnp.dot(xb, W1_ref[...], preferred_element_type=f32)
        + b1_ref[...], 0.0)
    h2 = jnp.maximum(
        jnp.dot(h1.astype(bf), W2_ref[...], preferred_element_type=f32)
        + b2_ref[...], 0.0)
    e8 = jnp.dot(h2.astype(bf), W3_ref[...], preferred_element_type=f32)
    e = e8[:, 0:1] + b3_ref[...]

    g = jnp.maximum(
        jnp.dot(xb, Ws1_ref[...], preferred_element_type=f32)
        + bs1_ref[...], 0.0)
    logits = jnp.dot(g.astype(bf), Ws2_ref[...], preferred_element_type=f32) \
        + bs2_ref[...]
    lmax = jnp.max(logits, axis=1, keepdims=True)
    cio = jax.lax.broadcasted_iota(jnp.int32, logits.shape, 1)
    sid = jnp.min(jnp.where(logits == lmax, cio, logits.shape[1]),
                  axis=1, keepdims=True)

    diff = x - native_ref[0]
    nd = jnp.sqrt(jnp.sum(diff * diff, axis=1, keepdims=True) + 1e-12)

    e_ref[0] = e
    sid_ref[0] = sid
    nd_ref[0] = nd

    nf = jnp.float32(n)
    mean_e = jnp.sum(e) / nf
    ce = e - mean_e
    se2 = jnp.sum(ce * ce)
    rugged = jnp.sqrt(se2 / (nf - 1.0))
    max_e = jnp.max(e)

    mean_nd = jnp.sum(nd) / nf
    cnd = nd - mean_nd
    snd2 = jnp.sum(cnd * cnd)
    corr = jnp.sum(cnd * ce) / jnp.sqrt(se2 * snd2)

    rio = jax.lax.broadcasted_iota(jnp.int32, nd.shape, 0)
    nd_min = jnp.min(nd)
    amin = jnp.min(jnp.where(nd == nd_min, rio, n))
    native_e = jnp.min(jnp.where(rio == amin, e, _INF))
    funnel = max_e - native_e

    lane = jax.lax.broadcasted_iota(jnp.int32, (1, 128), 1)
    met = (jnp.where(lane == 0, funnel, 0.0)
           + jnp.where(lane == 1, rugged, 0.0)
           + jnp.where(lane == 2, corr, 0.0)
           + jnp.where(lane == 3, native_e, 0.0))
    met_ref[0] = met


def _pair_body(xrow_ref, xT_ref, e_ref, sq_ref, nd_ref,
               basin_ref, depth_ref, width_ref, frus_ref, *, R, K):
    nb = pl.program_id(1)
    r0 = nb * R

    xb = xrow_ref[0]
    xT = xT_ref[0]
    e_r = e_ref[0]
    sq_r = sq_ref[0]
    nd_r = nd_ref[0]
    n = e_r.shape[1]

    dots = jnp.dot(xb, xT, preferred_element_type=jnp.float32)
    sq_b = jnp.reshape(sq_ref[0, 0, pl.ds(r0, R)], (R, 1))
    d2 = sq_b + sq_r - 2.0 * dots
    d = jnp.sqrt(jnp.maximum(d2, 0.0) + 1e-12)

    colio = jax.lax.broadcasted_iota(jnp.int32, (R, n), 1)

    dm = d
    idx0 = None
    m = None
    for k in range(K):
        m = jnp.min(dm, axis=1, keepdims=True)
        idx = jnp.min(jnp.where(dm == m, colio, n), axis=1, keepdims=True)
        if k == 0:
            idx0 = idx
        dm = jnp.where(colio == idx, _INF, dm)
    thresh = m

    neigh_min = jnp.min(
        jnp.where((dm == _INF) & (colio != idx0), e_r, _INF),
        axis=1, keepdims=True)

    e_b = jnp.reshape(e_ref[0, 0, pl.ds(r0, R)], (R, 1))
    is_basin = e_b <= neigh_min

    mask = d < thresh
    depth = jnp.max(jnp.where(mask, e_r, _NEG_INF), axis=1, keepdims=True) - e_b
    cnt = jnp.sum(mask.astype(jnp.float32), axis=1, keepdims=True)
    wsum = jnp.sum(jnp.where(mask, d, 0.0), axis=1, keepdims=True)
    width = wsum / jnp.maximum(cnt, 1.0)

    basin_ref[0, 0] = jnp.reshape(is_basin.astype(jnp.int32), (R,))
    depth_ref[0, 0] = jnp.reshape(jnp.where(is_basin, depth, 0.0), (R,))
    width_ref[0, 0] = jnp.reshape(jnp.where(is_basin, width, 0.0), (R,))

    nd_b = jnp.reshape(nd_ref[0, 0, pl.ds(r0, R)], (R, 1))
    gidx = r0 + jax.lax.broadcasted_iota(jnp.int32, (R, 1), 0)
    gt = (nd_r > nd_b) | ((nd_r == nd_b) & (colio > gidx))
    m2 = jnp.min(jnp.where(gt, nd_r, _INF), axis=1, keepdims=True)
    idx2 = jnp.min(jnp.where(gt & (nd_r == m2), colio, n),
                   axis=1, keepdims=True)
    e_succ = jnp.min(jnp.where(colio == idx2, e_r, _INF),
                     axis=1, keepdims=True)
    contrib = ((e_succ > e_b) & (idx2 < n)).astype(jnp.float32)
    part = jnp.sum(contrib)

    @pl.when(nb == 0)
    def _():
        frus_ref[0, 0] = jnp.zeros((128,), dtype=jnp.float32)

    frus_ref[0, 0] = frus_ref[0, 0] + part


def kernel(conformations, native_state, W1, b1, W2, b2, W3, b3,
           Ws1, bs1, Ws2, bs2):
    B, N, D = conformations.shape
    H = W1.shape[1]
    H2 = W2.shape[1]
    K = 10
    R = 512
    NB = N // R

    f32 = jnp.float32
    bf = jnp.bfloat16
    conf = conformations.astype(f32)
    conf_bf = conf.astype(bf)
    confT_bf = jnp.transpose(conf_bf, (0, 2, 1))

    sq = jnp.sum(conf * conf, axis=-1)
    sq_row = jnp.reshape(sq, (B, 1, N))

    C = Ws2.shape[1]
    Ws2p = jnp.zeros((H, 8), f32).at[:, :C].set(Ws2).astype(bf)
    bs2p = jnp.full((1, 8), -1e30, f32).at[0, :C].set(bs2)
    W3p = jnp.zeros((H2, 8), f32).at[:, 0:1].set(W3).astype(bf)

    b1r = jnp.reshape(b1, (1, H))
    b2r = jnp.reshape(b2, (1, H2))
    b3r = jnp.reshape(b3, (1, 1))
    bs1r = jnp.reshape(bs1, (1, H))
    nat = jnp.reshape(native_state, (B, 1, D))

    W1b = W1.astype(bf)
    W2b = W2.astype(bf)
    Ws1b = Ws1.astype(bf)

    wspec = lambda *shape: pl.BlockSpec(shape, lambda b: (0,) * len(shape))

    e_col, sid_col, nd_col, met = pl.pallas_call(
        _prologue_body,
        grid=(B,),
        in_specs=[
            pl.BlockSpec((1, N, D), lambda b: (b, 0, 0)),
            pl.BlockSpec((1, N, D), lambda b: (b, 0, 0)),
            pl.BlockSpec((1, 1, D), lambda b: (b, 0, 0)),
            wspec(D, H), wspec(1, H),
            wspec(H, H2), wspec(1, H2),
            wspec(H2, 8), wspec(1, 1),
            wspec(D, H), wspec(1, H),
            wspec(H, 8), wspec(1, 8),
        ],
        out_specs=[
            pl.BlockSpec((1, N, 1), lambda b: (b, 0, 0)),
            pl.BlockSpec((1, N, 1), lambda b: (b, 0, 0)),
            pl.BlockSpec((1, N, 1), lambda b: (b, 0, 0)),
            pl.BlockSpec((1, 1, 128), lambda b: (b, 0, 0)),
        ],
        out_shape=[
            jax.ShapeDtypeStruct((B, N, 1), f32),
            jax.ShapeDtypeStruct((B, N, 1), jnp.int32),
            jax.ShapeDtypeStruct((B, N, 1), f32),
            jax.ShapeDtypeStruct((B, 1, 128), f32),
        ],
    )(conf, conf_bf, nat, W1b, b1r, W2b, b2r, W3p, b3r, Ws1b, bs1r,
      Ws2p, bs2p)

    e_row = jnp.reshape(e_col, (B, 1, N))
    nd_row = jnp.reshape(nd_col, (B, 1, N))

    basin, depth, width, frus = pl.pallas_call(
        functools.partial(_pair_body, R=R, K=K),
        grid=(B, NB),
        in_specs=[
            pl.BlockSpec((1, R, D), lambda b, nb: (b, nb, 0)),
            pl.BlockSpec((1, D, N), lambda b, nb: (b, 0, 0)),
            pl.BlockSpec((1, 1, N), lambda b, nb: (b, 0, 0)),
            pl.BlockSpec((1, 1, N), lambda b, nb: (b, 0, 0)),
            pl.BlockSpec((1, 1, N), lambda b, nb: (b, 0, 0)),
        ],
        out_specs=[
            pl.BlockSpec((1, 1, R), lambda b, nb: (b * NB + nb, 0, 0)),
            pl.BlockSpec((1, 1, R), lambda b, nb: (b * NB + nb, 0, 0)),
            pl.BlockSpec((1, 1, R), lambda b, nb: (b * NB + nb, 0, 0)),
            pl.BlockSpec((1, 1, 128), lambda b, nb: (b, 0, 0)),
        ],
        out_shape=[
            jax.ShapeDtypeStruct((B * NB, 1, R), jnp.int32),
            jax.ShapeDtypeStruct((B * NB, 1, R), f32),
            jax.ShapeDtypeStruct((B * NB, 1, R), f32),
            jax.ShapeDtypeStruct((B, 1, 128), f32),
        ],
    )(conf_bf, confT_bf, e_row, sq_row, nd_row)

    e_out = jnp.reshape(e_col, (B, N))
    is_basin = jnp.reshape(basin, (B, N)) != 0
    depth_out = jnp.reshape(depth, (B, N))
    width_out = jnp.reshape(width, (B, N))
    sid_out = jnp.reshape(sid_col, (B, N))

    frustration = frus[:, 0, 0] / jnp.float32(N - 1)
    metrics = jnp.stack(
        [met[:, 0, 0], met[:, 0, 1], met[:, 0, 2], frustration,
         met[:, 0, 3]], axis=1)

    return (e_out, is_basin, depth_out, width_out, sid_out, metrics)


# speedup vs baseline: 17.5876x; 1.2113x over previous
import functools

import jax
import jax.numpy as jnp
from jax.experimental import pallas as pl
from jax.experimental.pallas import tpu as pltpu

_INF = float("inf")
_NEG_INF = float("-inf")


def _prologue_body(conf_ref, confbf_ref, native_ref, W1_ref, b1_ref,
                   W2_ref, b2_ref, W3_ref, b3_ref, Ws1_ref, bs1_ref,
                   Ws2_ref, bs2_ref,
                   e_ref, sid_ref, nd_ref, met_ref):
    x = conf_ref[0]
    xb = confbf_ref[0]
    n = x.shape[0]
    bf = jnp.bfloat16
    f32 = jnp.float32

    h1 = jnp.maximum(
        jnp.dot(xb, W1_ref[...], preferred_element_type=f32)
        + b1_ref[...], 0.0)
    h2 = jnp.maximum(
        jnp.dot(h1.astype(bf), W2_ref[...], preferred_element_type=f32)
        + b2_ref[...], 0.0)
    e8 = jnp.dot(h2.astype(bf), W3_ref[...], preferred_element_type=f32)
    e = e8[:, 0:1] + b3_ref[...]

    g = jnp.maximum(
        jnp.dot(xb, Ws1_ref[...], preferred_element_type=f32)
        + bs1_ref[...], 0.0)
    logits = jnp.dot(g.astype(bf), Ws2_ref[...], preferred_element_type=f32) \
        + bs2_ref[...]
    lmax = jnp.max(logits, axis=1, keepdims=True)
    cio = jax.lax.broadcasted_iota(jnp.int32, logits.shape, 1)
    sid = jnp.min(jnp.where(logits == lmax, cio, logits.shape[1]),
                  axis=1, keepdims=True)

    diff = x - native_ref[0]
    nd = jnp.sqrt(jnp.sum(diff * diff, axis=1, keepdims=True) + 1e-12)

    e_ref[0] = e
    sid_ref[0] = sid
    nd_ref[0] = nd

    nf = jnp.float32(n)
    mean_e = jnp.sum(e) / nf
    ce = e - mean_e
    se2 = jnp.sum(ce * ce)
    rugged = jnp.sqrt(se2 / (nf - 1.0))
    max_e = jnp.max(e)

    mean_nd = jnp.sum(nd) / nf
    cnd = nd - mean_nd
    snd2 = jnp.sum(cnd * cnd)
    corr = jnp.sum(cnd * ce) / jnp.sqrt(se2 * snd2)

    rio = jax.lax.broadcasted_iota(jnp.int32, nd.shape, 0)
    nd_min = jnp.min(nd)
    amin = jnp.min(jnp.where(nd == nd_min, rio, n))
    native_e = jnp.min(jnp.where(rio == amin, e, _INF))
    funnel = max_e - native_e

    lane = jax.lax.broadcasted_iota(jnp.int32, (1, 128), 1)
    met = (jnp.where(lane == 0, funnel, 0.0)
           + jnp.where(lane == 1, rugged, 0.0)
           + jnp.where(lane == 2, corr, 0.0)
           + jnp.where(lane == 3, native_e, 0.0))
    met_ref[0] = met


def _pair_body(xrow_ref, xT_ref, e_ref, sq_ref, nd_ref,
               basin_ref, depth_ref, width_ref, frus_ref, *, R, K):
    nb = pl.program_id(1)
    r0 = nb * R

    xb = xrow_ref[0]
    xT = xT_ref[0]
    e_r = e_ref[0]
    sq_r = sq_ref[0]
    nd_r = nd_ref[0]
    n = e_r.shape[1]

    dots = jnp.dot(xb, xT, preferred_element_type=jnp.float32)
    sq_b = jnp.reshape(sq_ref[0, 0, pl.ds(r0, R)], (R, 1))
    d2 = sq_b + sq_r - 2.0 * dots
    d = jnp.sqrt(jnp.maximum(d2, 0.0) + 1e-12)

    colio = jax.lax.broadcasted_iota(jnp.int32, (R, n), 1)
    e_b = jnp.reshape(e_ref[0, 0, pl.ds(r0, R)], (R, 1))

    def _stats(thresh, neigh_min):
        is_basin = e_b <= neigh_min
        mask = d < thresh
        depth = jnp.max(jnp.where(mask, e_r, _NEG_INF), axis=1,
                        keepdims=True) - e_b
        cnt = jnp.sum(mask.astype(jnp.float32), axis=1, keepdims=True)
        wsum = jnp.sum(jnp.where(mask, d, 0.0), axis=1, keepdims=True)
        width = wsum / jnp.maximum(cnt, 1.0)
        basin_ref[0, 0] = jnp.reshape(is_basin.astype(jnp.int32), (R,))
        depth_ref[0, 0] = jnp.reshape(jnp.where(is_basin, depth, 0.0), (R,))
        width_ref[0, 0] = jnp.reshape(jnp.where(is_basin, width, 0.0), (R,))

    dm = d
    m = jnp.min(dm, axis=1, keepdims=True)
    idx0 = jnp.min(jnp.where(dm == m, colio, n), axis=1, keepdims=True)
    for _ in range(K - 1):
        dm = jnp.where(dm == m, _INF, dm)
        m = jnp.min(dm, axis=1, keepdims=True)
    dm = jnp.where(dm == m, _INF, dm)
    thresh = m
    sel = dm == _INF
    total = jnp.sum(sel.astype(jnp.float32), axis=1, keepdims=True)
    neigh_min = jnp.min(jnp.where(sel & (colio != idx0), e_r, _INF),
                        axis=1, keepdims=True)
    _stats(thresh, neigh_min)

    exact_needed = jnp.any(total != jnp.float32(K))

    @pl.when(exact_needed)
    def _():
        dm2 = d
        idx0s = None
        ms = None
        for k in range(K):
            ms = jnp.min(dm2, axis=1, keepdims=True)
            idxs = jnp.min(jnp.where(dm2 == ms, colio, n), axis=1,
                           keepdims=True)
            if k == 0:
                idx0s = idxs
            dm2 = jnp.where(colio == idxs, _INF, dm2)
        nm = jnp.min(jnp.where((dm2 == _INF) & (colio != idx0s), e_r, _INF),
                     axis=1, keepdims=True)
        _stats(ms, nm)

    nd_b = jnp.reshape(nd_ref[0, 0, pl.ds(r0, R)], (R, 1))
    gidx = r0 + jax.lax.broadcasted_iota(jnp.int32, (R, 1), 0)
    gt = (nd_r > nd_b) | ((nd_r == nd_b) & (colio > gidx))
    m2 = jnp.min(jnp.where(gt, nd_r, _INF), axis=1, keepdims=True)
    idx2 = jnp.min(jnp.where(gt & (nd_r == m2), colio, n),
                   axis=1, keepdims=True)
    e_succ = jnp.min(jnp.where(colio == idx2, e_r, _INF),
                     axis=1, keepdims=True)
    contrib = ((e_succ > e_b) & (idx2 < n)).astype(jnp.float32)
    part = jnp.sum(contrib)

    @pl.when(nb == 0)
    def _():
        frus_ref[0, 0] = jnp.zeros((128,), dtype=jnp.float32)

    frus_ref[0, 0] = frus_ref[0, 0] + part


def kernel(conformations, native_state, W1, b1, W2, b2, W3, b3,
           Ws1, bs1, Ws2, bs2):
    B, N, D = conformations.shape
    H = W1.shape[1]
    H2 = W2.shape[1]
    K = 10
    R = 512
    NB = N // R

    f32 = jnp.float32
    bf = jnp.bfloat16
    conf = conformations.astype(f32)
    conf_bf = conf.astype(bf)
    confT_bf = jnp.transpose(conf_bf, (0, 2, 1))

    sq = jnp.sum(conf * conf, axis=-1)
    sq_row = jnp.reshape(sq, (B, 1, N))

    C = Ws2.shape[1]
    Ws2p = jnp.zeros((H, 8), f32).at[:, :C].set(Ws2).astype(bf)
    bs2p = jnp.full((1, 8), -1e30, f32).at[0, :C].set(bs2)
    W3p = jnp.zeros((H2, 8), f32).at[:, 0:1].set(W3).astype(bf)

    b1r = jnp.reshape(b1, (1, H))
    b2r = jnp.reshape(b2, (1, H2))
    b3r = jnp.reshape(b3, (1, 1))
    bs1r = jnp.reshape(bs1, (1, H))
    nat = jnp.reshape(native_state, (B, 1, D))

    W1b = W1.astype(bf)
    W2b = W2.astype(bf)
    Ws1b = Ws1.astype(bf)

    wspec = lambda *shape: pl.BlockSpec(shape, lambda b: (0,) * len(shape))

    e_col, sid_col, nd_col, met = pl.pallas_call(
        _prologue_body,
        grid=(B,),
        in_specs=[
            pl.BlockSpec((1, N, D), lambda b: (b, 0, 0)),
            pl.BlockSpec((1, N, D), lambda b: (b, 0, 0)),
            pl.BlockSpec((1, 1, D), lambda b: (b, 0, 0)),
            wspec(D, H), wspec(1, H),
            wspec(H, H2), wspec(1, H2),
            wspec(H2, 8), wspec(1, 1),
            wspec(D, H), wspec(1, H),
            wspec(H, 8), wspec(1, 8),
        ],
        out_specs=[
            pl.BlockSpec((1, N, 1), lambda b: (b, 0, 0)),
            pl.BlockSpec((1, N, 1), lambda b: (b, 0, 0)),
            pl.BlockSpec((1, N, 1), lambda b: (b, 0, 0)),
            pl.BlockSpec((1, 1, 128), lambda b: (b, 0, 0)),
        ],
        out_shape=[
            jax.ShapeDtypeStruct((B, N, 1), f32),
            jax.ShapeDtypeStruct((B, N, 1), jnp.int32),
            jax.ShapeDtypeStruct((B, N, 1), f32),
            jax.ShapeDtypeStruct((B, 1, 128), f32),
        ],
    )(conf, conf_bf, nat, W1b, b1r, W2b, b2r, W3p, b3r, Ws1b, bs1r,
      Ws2p, bs2p)

    e_row = jnp.reshape(e_col, (B, 1, N))
    nd_row = jnp.reshape(nd_col, (B, 1, N))

    basin, depth, width, frus = pl.pallas_call(
        functools.partial(_pair_body, R=R, K=K),
        grid=(B, NB),
        in_specs=[
            pl.BlockSpec((1, R, D), lambda b, nb: (b, nb, 0)),
            pl.BlockSpec((1, D, N), lambda b, nb: (b, 0, 0)),
            pl.BlockSpec((1, 1, N), lambda b, nb: (b, 0, 0)),
            pl.BlockSpec((1, 1, N), lambda b, nb: (b, 0, 0)),
            pl.BlockSpec((1, 1, N), lambda b, nb: (b, 0, 0)),
        ],
        out_specs=[
            pl.BlockSpec((1, 1, R), lambda b, nb: (b * NB + nb, 0, 0)),
            pl.BlockSpec((1, 1, R), lambda b, nb: (b * NB + nb, 0, 0)),
            pl.BlockSpec((1, 1, R), lambda b, nb: (b * NB + nb, 0, 0)),
            pl.BlockSpec((1, 1, 128), lambda b, nb: (b, 0, 0)),
        ],
        out_shape=[
            jax.ShapeDtypeStruct((B * NB, 1, R), jnp.int32),
            jax.ShapeDtypeStruct((B * NB, 1, R), f32),
            jax.ShapeDtypeStruct((B * NB, 1, R), f32),
            jax.ShapeDtypeStruct((B, 1, 128), f32),
        ],
    )(conf_bf, confT_bf, e_row, sq_row, nd_row)

    e_out = jnp.reshape(e_col, (B, N))
    is_basin = jnp.reshape(basin, (B, N)) != 0
    depth_out = jnp.reshape(depth, (B, N))
    width_out = jnp.reshape(width, (B, N))
    sid_out = jnp.reshape(sid_col, (B, N))

    frustration = frus[:, 0, 0] / jnp.float32(N - 1)
    metrics = jnp.stack(
        [met[:, 0, 0], met[:, 0, 1], met[:, 0, 2], frustration,
         met[:, 0, 3]], axis=1)

    return (e_out, is_basin, depth_out, width_out, sid_out, metrics)


# speedup vs baseline: 18.8061x; 1.0693x over previous
import functools

import jax
import jax.numpy as jnp
from jax.experimental import pallas as pl
from jax.experimental.pallas import tpu as pltpu

_INF = float("inf")
_NEG_INF = float("-inf")


def _prologue_body(conf_ref, confbf_ref, native_ref, W1_ref, b1_ref,
                   W2_ref, b2_ref, W3_ref, b3_ref, Ws1_ref, bs1_ref,
                   Ws2_ref, bs2_ref,
                   e_ref, sid_ref, nd_ref, met_ref):
    x = conf_ref[0]
    xb = confbf_ref[0]
    n = x.shape[0]
    bf = jnp.bfloat16
    f32 = jnp.float32

    h1 = jnp.maximum(
        jnp.dot(xb, W1_ref[...], preferred_element_type=f32)
        + b1_ref[...], 0.0)
    h2 = jnp.maximum(
        jnp.dot(h1.astype(bf), W2_ref[...], preferred_element_type=f32)
        + b2_ref[...], 0.0)
    e8 = jnp.dot(h2.astype(bf), W3_ref[...], preferred_element_type=f32)
    e = e8[:, 0:1] + b3_ref[...]

    g = jnp.maximum(
        jnp.dot(xb, Ws1_ref[...], preferred_element_type=f32)
        + bs1_ref[...], 0.0)
    logits = jnp.dot(g.astype(bf), Ws2_ref[...], preferred_element_type=f32) \
        + bs2_ref[...]
    lmax = jnp.max(logits, axis=1, keepdims=True)
    cio = jax.lax.broadcasted_iota(jnp.int32, logits.shape, 1)
    sid = jnp.min(jnp.where(logits == lmax, cio, logits.shape[1]),
                  axis=1, keepdims=True)

    diff = x - native_ref[0]
    nd = jnp.sqrt(jnp.sum(diff * diff, axis=1, keepdims=True) + 1e-12)

    e_ref[0] = e
    sid_ref[0] = sid
    nd_ref[0] = nd

    nf = jnp.float32(n)
    mean_e = jnp.sum(e) / nf
    ce = e - mean_e
    se2 = jnp.sum(ce * ce)
    rugged = jnp.sqrt(se2 / (nf - 1.0))
    max_e = jnp.max(e)

    mean_nd = jnp.sum(nd) / nf
    cnd = nd - mean_nd
    snd2 = jnp.sum(cnd * cnd)
    corr = jnp.sum(cnd * ce) / jnp.sqrt(se2 * snd2)

    rio = jax.lax.broadcasted_iota(jnp.int32, nd.shape, 0)
    nd_min = jnp.min(nd)
    amin = jnp.min(jnp.where(nd == nd_min, rio, n))
    native_e = jnp.min(jnp.where(rio == amin, e, _INF))
    funnel = max_e - native_e

    lane = jax.lax.broadcasted_iota(jnp.int32, (1, 128), 1)
    met = (jnp.where(lane == 0, funnel, 0.0)
           + jnp.where(lane == 1, rugged, 0.0)
           + jnp.where(lane == 2, corr, 0.0)
           + jnp.where(lane == 3, native_e, 0.0))
    met_ref[0] = met


def _pair_body(xrow_ref, xT_ref, e_ref, sq_ref, nd_ref,
               basin_ref, depth_ref, width_ref, frus_ref, *, R, K):
    nb = pl.program_id(1)
    r0 = nb * R

    xb = xrow_ref[0]
    xT = xT_ref[0]
    e_r = e_ref[0]
    sq_r = sq_ref[0]
    nd_r = nd_ref[0]
    n = e_r.shape[1]

    dots = jnp.dot(xb, xT, preferred_element_type=jnp.float32)
    sq_b = jnp.reshape(sq_ref[0, 0, pl.ds(r0, R)], (R, 1))
    d2 = sq_b + sq_r - 2.0 * dots
    d = jnp.sqrt(jnp.maximum(d2, 0.0) + 1e-12)

    colio = jax.lax.broadcasted_iota(jnp.int32, (R, n), 1)
    e_b = jnp.reshape(e_ref[0, 0, pl.ds(r0, R)], (R, 1))

    def _stats(thresh, neigh_min):
        is_basin = e_b <= neigh_min
        mask = d < thresh
        depth = jnp.max(jnp.where(mask, e_r, _NEG_INF), axis=1,
                        keepdims=True) - e_b
        cnt = jnp.sum(mask.astype(jnp.float32), axis=1, keepdims=True)
        wsum = jnp.sum(jnp.where(mask, d, 0.0), axis=1, keepdims=True)
        width = wsum / jnp.maximum(cnt, 1.0)
        basin_ref[0, 0] = jnp.reshape(is_basin.astype(jnp.int32), (R,))
        depth_ref[0, 0] = jnp.reshape(jnp.where(is_basin, depth, 0.0), (R,))
        width_ref[0, 0] = jnp.reshape(jnp.where(is_basin, width, 0.0), (R,))

    dm = d
    m0 = jnp.min(dm, axis=1, keepdims=True)
    m = m0
    for _ in range(K - 1):
        dm = jnp.where(dm == m, _INF, dm)
        m = jnp.min(dm, axis=1, keepdims=True)
    dm = jnp.where(dm == m, _INF, dm)
    thresh = m
    sel = dm == _INF
    total = jnp.sum(sel.astype(jnp.float32), axis=1, keepdims=True)
    neigh_min = jnp.min(jnp.where(sel & (d != m0), e_r, _INF),
                        axis=1, keepdims=True)
    _stats(thresh, neigh_min)

    exact_needed = jnp.any(total != jnp.float32(K))

    @pl.when(exact_needed)
    def _():
        dm2 = d
        idx0s = None
        ms = None
        for k in range(K):
            ms = jnp.min(dm2, axis=1, keepdims=True)
            idxs = jnp.min(jnp.where(dm2 == ms, colio, n), axis=1,
                           keepdims=True)
            if k == 0:
                idx0s = idxs
            dm2 = jnp.where(colio == idxs, _INF, dm2)
        nm = jnp.min(jnp.where((dm2 == _INF) & (colio != idx0s), e_r, _INF),
                     axis=1, keepdims=True)
        _stats(ms, nm)

    nd_b = jnp.reshape(nd_ref[0, 0, pl.ds(r0, R)], (R, 1))
    gidx = r0 + jax.lax.broadcasted_iota(jnp.int32, (R, 1), 0)
    gt = (nd_r > nd_b) | ((nd_r == nd_b) & (colio > gidx))
    m2 = jnp.min(jnp.where(gt, nd_r, _INF), axis=1, keepdims=True)
    e_succ = jnp.min(jnp.where(gt & (nd_r == m2), e_r, _INF),
                     axis=1, keepdims=True)
    contrib = ((e_succ > e_b) & (m2 < _INF)).astype(jnp.float32)
    part = jnp.sum(contrib)

    @pl.when(nb == 0)
    def _():
        frus_ref[0, 0] = jnp.zeros((128,), dtype=jnp.float32)

    frus_ref[0, 0] = frus_ref[0, 0] + part


def kernel(conformations, native_state, W1, b1, W2, b2, W3, b3,
           Ws1, bs1, Ws2, bs2):
    B, N, D = conformations.shape
    H = W1.shape[1]
    H2 = W2.shape[1]
    K = 10
    R = 512
    NB = N // R

    f32 = jnp.float32
    bf = jnp.bfloat16
    conf = conformations.astype(f32)
    conf_bf = conf.astype(bf)
    confT_bf = jnp.transpose(conf_bf, (0, 2, 1))

    sq = jnp.sum(conf * conf, axis=-1)
    sq_row = jnp.reshape(sq, (B, 1, N))

    C = Ws2.shape[1]
    Ws2p = jnp.zeros((H, 8), f32).at[:, :C].set(Ws2).astype(bf)
    bs2p = jnp.full((1, 8), -1e30, f32).at[0, :C].set(bs2)
    W3p = jnp.zeros((H2, 8), f32).at[:, 0:1].set(W3).astype(bf)

    b1r = jnp.reshape(b1, (1, H))
    b2r = jnp.reshape(b2, (1, H2))
    b3r = jnp.reshape(b3, (1, 1))
    bs1r = jnp.reshape(bs1, (1, H))
    nat = jnp.reshape(native_state, (B, 1, D))

    W1b = W1.astype(bf)
    W2b = W2.astype(bf)
    Ws1b = Ws1.astype(bf)

    wspec = lambda *shape: pl.BlockSpec(shape, lambda b: (0,) * len(shape))

    e_col, sid_col, nd_col, met = pl.pallas_call(
        _prologue_body,
        grid=(B,),
        in_specs=[
            pl.BlockSpec((1, N, D), lambda b: (b, 0, 0)),
            pl.BlockSpec((1, N, D), lambda b: (b, 0, 0)),
            pl.BlockSpec((1, 1, D), lambda b: (b, 0, 0)),
            wspec(D, H), wspec(1, H),
            wspec(H, H2), wspec(1, H2),
            wspec(H2, 8), wspec(1, 1),
            wspec(D, H), wspec(1, H),
            wspec(H, 8), wspec(1, 8),
        ],
        out_specs=[
            pl.BlockSpec((1, N, 1), lambda b: (b, 0, 0)),
            pl.BlockSpec((1, N, 1), lambda b: (b, 0, 0)),
            pl.BlockSpec((1, N, 1), lambda b: (b, 0, 0)),
            pl.BlockSpec((1, 1, 128), lambda b: (b, 0, 0)),
        ],
        out_shape=[
            jax.ShapeDtypeStruct((B, N, 1), f32),
            jax.ShapeDtypeStruct((B, N, 1), jnp.int32),
            jax.ShapeDtypeStruct((B, N, 1), f32),
            jax.ShapeDtypeStruct((B, 1, 128), f32),
        ],
    )(conf, conf_bf, nat, W1b, b1r, W2b, b2r, W3p, b3r, Ws1b, bs1r,
      Ws2p, bs2p)

    e_row = jnp.reshape(e_col, (B, 1, N))
    nd_row = jnp.reshape(nd_col, (B, 1, N))

    basin, depth, width, frus = pl.pallas_call(
        functools.partial(_pair_body, R=R, K=K),
        grid=(B, NB),
        in_specs=[
            pl.BlockSpec((1, R, D), lambda b, nb: (b, nb, 0)),
            pl.BlockSpec((1, D, N), lambda b, nb: (b, 0, 0)),
            pl.BlockSpec((1, 1, N), lambda b, nb: (b, 0, 0)),
            pl.BlockSpec((1, 1, N), lambda b, nb: (b, 0, 0)),
            pl.BlockSpec((1, 1, N), lambda b, nb: (b, 0, 0)),
        ],
        out_specs=[
            pl.BlockSpec((1, 1, R), lambda b, nb: (b * NB + nb, 0, 0)),
            pl.BlockSpec((1, 1, R), lambda b, nb: (b * NB + nb, 0, 0)),
            pl.BlockSpec((1, 1, R), lambda b, nb: (b * NB + nb, 0, 0)),
            pl.BlockSpec((1, 1, 128), lambda b, nb: (b, 0, 0)),
        ],
        out_shape=[
            jax.ShapeDtypeStruct((B * NB, 1, R), jnp.int32),
            jax.ShapeDtypeStruct((B * NB, 1, R), f32),
            jax.ShapeDtypeStruct((B * NB, 1, R), f32),
            jax.ShapeDtypeStruct((B, 1, 128), f32),
        ],
    )(conf_bf, confT_bf, e_row, sq_row, nd_row)

    e_out = jnp.reshape(e_col, (B, N))
    is_basin = jnp.reshape(basin, (B, N)) != 0
    depth_out = jnp.reshape(depth, (B, N))
    width_out = jnp.reshape(width, (B, N))
    sid_out = jnp.reshape(sid_col, (B, N))

    frustration = frus[:, 0, 0] / jnp.float32(N - 1)
    metrics = jnp.stack(
        [met[:, 0, 0], met[:, 0, 1], met[:, 0, 2], frustration,
         met[:, 0, 3]], axis=1)

    return (e_out, is_basin, depth_out, width_out, sid_out, metrics)


# speedup vs baseline: 24.9261x; 1.3254x over previous
import functools

import jax
import jax.numpy as jnp
from jax.experimental import pallas as pl
from jax.experimental.pallas import tpu as pltpu

_INF = float("inf")
_NEG_INF = float("-inf")


def _prologue_body(conf_ref, confbf_ref, native_ref, W1_ref, b1_ref,
                   W2_ref, b2_ref, W3_ref, b3_ref, Ws1_ref, bs1_ref,
                   Ws2_ref, bs2_ref,
                   e_ref, sid_ref, nd_ref, met_ref):
    x = conf_ref[0]
    xb = confbf_ref[0]
    n = x.shape[0]
    bf = jnp.bfloat16
    f32 = jnp.float32

    h1 = jnp.maximum(
        jnp.dot(xb, W1_ref[...], preferred_element_type=f32)
        + b1_ref[...], 0.0)
    h2 = jnp.maximum(
        jnp.dot(h1.astype(bf), W2_ref[...], preferred_element_type=f32)
        + b2_ref[...], 0.0)
    e8 = jnp.dot(h2.astype(bf), W3_ref[...], preferred_element_type=f32)
    e = e8[:, 0:1] + b3_ref[...]

    g = jnp.maximum(
        jnp.dot(xb, Ws1_ref[...], preferred_element_type=f32)
        + bs1_ref[...], 0.0)
    logits = jnp.dot(g.astype(bf), Ws2_ref[...], preferred_element_type=f32) \
        + bs2_ref[...]
    lmax = jnp.max(logits, axis=1, keepdims=True)
    cio = jax.lax.broadcasted_iota(jnp.int32, logits.shape, 1)
    sid = jnp.min(jnp.where(logits == lmax, cio, logits.shape[1]),
                  axis=1, keepdims=True)

    diff = x - native_ref[0]
    nd = jnp.sqrt(jnp.sum(diff * diff, axis=1, keepdims=True) + 1e-12)

    e_ref[0] = e
    sid_ref[0] = sid
    nd_ref[0] = nd

    nf = jnp.float32(n)
    mean_e = jnp.sum(e) / nf
    ce = e - mean_e
    se2 = jnp.sum(ce * ce)
    rugged = jnp.sqrt(se2 / (nf - 1.0))
    max_e = jnp.max(e)

    mean_nd = jnp.sum(nd) / nf
    cnd = nd - mean_nd
    snd2 = jnp.sum(cnd * cnd)
    corr = jnp.sum(cnd * ce) / jnp.sqrt(se2 * snd2)

    rio = jax.lax.broadcasted_iota(jnp.int32, nd.shape, 0)
    nd_min = jnp.min(nd)
    amin = jnp.min(jnp.where(nd == nd_min, rio, n))
    native_e = jnp.min(jnp.where(rio == amin, e, _INF))
    funnel = max_e - native_e

    lane = jax.lax.broadcasted_iota(jnp.int32, (1, 128), 1)
    met = (jnp.where(lane == 0, funnel, 0.0)
           + jnp.where(lane == 1, rugged, 0.0)
           + jnp.where(lane == 2, corr, 0.0)
           + jnp.where(lane == 3, native_e, 0.0))
    met_ref[0] = met


def _pair_body(xrow_ref, xT_ref, e_ref, sq_ref, nd_ref,
               basin_ref, depth_ref, width_ref, frus_ref, *, R, K):
    nb = pl.program_id(1)
    r0 = nb * R

    xb = xrow_ref[0]
    xT = xT_ref[0]
    e_r = e_ref[0]
    sq_r = sq_ref[0]
    nd_r = nd_ref[0]
    n = e_r.shape[1]

    dots = jnp.dot(xb, xT, preferred_element_type=jnp.float32)
    sq_b = jnp.reshape(sq_ref[0, 0, pl.ds(r0, R)], (R, 1))
    d2 = sq_b + sq_r - 2.0 * dots
    d = jnp.sqrt(jnp.maximum(d2, 0.0) + 1e-12)

    e_b = jnp.reshape(e_ref[0, 0, pl.ds(r0, R)], (R, 1))

    def _stats(thresh, neigh_min):
        is_basin = e_b <= neigh_min
        mask = d < thresh
        depth = jnp.max(jnp.where(mask, e_r, _NEG_INF), axis=1,
                        keepdims=True) - e_b
        cnt = jnp.sum(mask.astype(jnp.float32), axis=1, keepdims=True)
        wsum = jnp.sum(jnp.where(mask, d, 0.0), axis=1, keepdims=True)
        width = wsum / jnp.maximum(cnt, 1.0)
        basin_ref[0, 0] = jnp.reshape(is_basin.astype(jnp.int32), (R,))
        depth_ref[0, 0] = jnp.reshape(jnp.where(is_basin, depth, 0.0), (R,))
        width_ref[0, 0] = jnp.reshape(jnp.where(is_basin, width, 0.0), (R,))

    f32 = jnp.float32
    s1 = jnp.full((R, 128), _INF, f32)
    s2 = jnp.full((R, 128), _INF, f32)
    s3 = jnp.full((R, 128), _INF, f32)
    s4 = jnp.full((R, 128), _INF, f32)
    for g in range(n // 128):
        v = d[:, g * 128:(g + 1) * 128]
        w1 = jnp.maximum(s1, v)
        s1 = jnp.minimum(s1, v)
        w2 = jnp.maximum(s2, w1)
        s2 = jnp.minimum(s2, w1)
        w3 = jnp.maximum(s3, w2)
        s3 = jnp.minimum(s3, w2)
        s4 = jnp.minimum(s4, w3)
    cand = jnp.concatenate([s1, s2, s3, s4], axis=1)

    cm = cand
    m0 = jnp.min(cm, axis=1, keepdims=True)
    m = m0
    for _ in range(K - 1):
        cm = jnp.where(cm == m, _INF, cm)
        m = jnp.min(cm, axis=1, keepdims=True)
    cm = jnp.where(cm == m, _INF, cm)
    thresh = m
    total = jnp.sum((cm == _INF).astype(f32), axis=1, keepdims=True)

    le = d <= thresh
    cnt_le = jnp.sum(le.astype(f32), axis=1, keepdims=True)
    neigh_min = jnp.min(jnp.where(le & (d != m0), e_r, _INF),
                        axis=1, keepdims=True)
    _stats(thresh, neigh_min)

    lane_bad = jnp.any(s4 <= thresh, axis=1, keepdims=True)
    bad = (total != f32(K)) | (cnt_le != f32(K)) | lane_bad
    exact_needed = jnp.any(bad)

    @pl.when(exact_needed)
    def _():
        colio = jax.lax.broadcasted_iota(jnp.int32, (R, n), 1)
        dm2 = d
        idx0s = None
        ms = None
        for k in range(K):
            ms = jnp.min(dm2, axis=1, keepdims=True)
            idxs = jnp.min(jnp.where(dm2 == ms, colio, n), axis=1,
                           keepdims=True)
            if k == 0:
                idx0s = idxs
            dm2 = jnp.where(colio == idxs, _INF, dm2)
        nm = jnp.min(jnp.where((dm2 == _INF) & (colio != idx0s), e_r, _INF),
                     axis=1, keepdims=True)
        _stats(ms, nm)

    nd_b = jnp.reshape(nd_ref[0, 0, pl.ds(r0, R)], (R, 1))
    gt = nd_r > nd_b
    m2 = jnp.min(jnp.where(gt, nd_r, _INF), axis=1, keepdims=True)
    e_succ = jnp.min(jnp.where(gt & (nd_r == m2), e_r, _INF),
                     axis=1, keepdims=True)
    contrib = ((e_succ > e_b) & (m2 < _INF)).astype(jnp.float32)
    part = jnp.sum(contrib)

    @pl.when(nb == 0)
    def _():
        frus_ref[0, 0] = jnp.zeros((128,), dtype=jnp.float32)

    frus_ref[0, 0] = frus_ref[0, 0] + part


def kernel(conformations, native_state, W1, b1, W2, b2, W3, b3,
           Ws1, bs1, Ws2, bs2):
    B, N, D = conformations.shape
    H = W1.shape[1]
    H2 = W2.shape[1]
    K = 10
    R = 512
    NB = N // R

    f32 = jnp.float32
    bf = jnp.bfloat16
    conf = conformations.astype(f32)
    conf_bf = conf.astype(bf)
    confT_bf = jnp.transpose(conf_bf, (0, 2, 1))

    sq = jnp.sum(conf * conf, axis=-1)
    sq_row = jnp.reshape(sq, (B, 1, N))

    C = Ws2.shape[1]
    Ws2p = jnp.zeros((H, 8), f32).at[:, :C].set(Ws2).astype(bf)
    bs2p = jnp.full((1, 8), -1e30, f32).at[0, :C].set(bs2)
    W3p = jnp.zeros((H2, 8), f32).at[:, 0:1].set(W3).astype(bf)

    b1r = jnp.reshape(b1, (1, H))
    b2r = jnp.reshape(b2, (1, H2))
    b3r = jnp.reshape(b3, (1, 1))
    bs1r = jnp.reshape(bs1, (1, H))
    nat = jnp.reshape(native_state, (B, 1, D))

    W1b = W1.astype(bf)
    W2b = W2.astype(bf)
    Ws1b = Ws1.astype(bf)

    wspec = lambda *shape: pl.BlockSpec(shape, lambda b: (0,) * len(shape))

    e_col, sid_col, nd_col, met = pl.pallas_call(
        _prologue_body,
        grid=(B,),
        in_specs=[
            pl.BlockSpec((1, N, D), lambda b: (b, 0, 0)),
            pl.BlockSpec((1, N, D), lambda b: (b, 0, 0)),
            pl.BlockSpec((1, 1, D), lambda b: (b, 0, 0)),
            wspec(D, H), wspec(1, H),
            wspec(H, H2), wspec(1, H2),
            wspec(H2, 8), wspec(1, 1),
            wspec(D, H), wspec(1, H),
            wspec(H, 8), wspec(1, 8),
        ],
        out_specs=[
            pl.BlockSpec((1, N, 1), lambda b: (b, 0, 0)),
            pl.BlockSpec((1, N, 1), lambda b: (b, 0, 0)),
            pl.BlockSpec((1, N, 1), lambda b: (b, 0, 0)),
            pl.BlockSpec((1, 1, 128), lambda b: (b, 0, 0)),
        ],
        out_shape=[
            jax.ShapeDtypeStruct((B, N, 1), f32),
            jax.ShapeDtypeStruct((B, N, 1), jnp.int32),
            jax.ShapeDtypeStruct((B, N, 1), f32),
            jax.ShapeDtypeStruct((B, 1, 128), f32),
        ],
    )(conf, conf_bf, nat, W1b, b1r, W2b, b2r, W3p, b3r, Ws1b, bs1r,
      Ws2p, bs2p)

    e_row = jnp.reshape(e_col, (B, 1, N))
    nd_row = jnp.reshape(nd_col, (B, 1, N))

    basin, depth, width, frus = pl.pallas_call(
        functools.partial(_pair_body, R=R, K=K),
        grid=(B, NB),
        in_specs=[
            pl.BlockSpec((1, R, D), lambda b, nb: (b, nb, 0)),
            pl.BlockSpec((1, D, N), lambda b, nb: (b, 0, 0)),
            pl.BlockSpec((1, 1, N), lambda b, nb: (b, 0, 0)),
            pl.BlockSpec((1, 1, N), lambda b, nb: (b, 0, 0)),
            pl.BlockSpec((1, 1, N), lambda b, nb: (b, 0, 0)),
        ],
        out_specs=[
            pl.BlockSpec((1, 1, R), lambda b, nb: (b * NB + nb, 0, 0)),
            pl.BlockSpec((1, 1, R), lambda b, nb: (b * NB + nb, 0, 0)),
            pl.BlockSpec((1, 1, R), lambda b, nb: (b * NB + nb, 0, 0)),
            pl.BlockSpec((1, 1, 128), lambda b, nb: (b, 0, 0)),
        ],
        out_shape=[
            jax.ShapeDtypeStruct((B * NB, 1, R), jnp.int32),
            jax.ShapeDtypeStruct((B * NB, 1, R), f32),
            jax.ShapeDtypeStruct((B * NB, 1, R), f32),
            jax.ShapeDtypeStruct((B, 1, 128), f32),
        ],
    )(conf_bf, confT_bf, e_row, sq_row, nd_row)

    e_out = jnp.reshape(e_col, (B, N))
    is_basin = jnp.reshape(basin, (B, N)) != 0
    depth_out = jnp.reshape(depth, (B, N))
    width_out = jnp.reshape(width, (B, N))
    sid_out = jnp.reshape(sid_col, (B, N))

    frustration = frus[:, 0, 0] / jnp.float32(N - 1)
    metrics = jnp.stack(
        [met[:, 0, 0], met[:, 0, 1], met[:, 0, 2], frustration,
         met[:, 0, 3]], axis=1)

    return (e_out, is_basin, depth_out, width_out, sid_out, metrics)


# speedup vs baseline: 24.9506x; 1.0010x over previous
import functools

import jax
import jax.numpy as jnp
from jax.experimental import pallas as pl
from jax.experimental.pallas import tpu as pltpu

_INF = float("inf")
_NEG_INF = float("-inf")


def _prologue_body(conf_ref, confbf_ref, native_ref, W1_ref, b1_ref,
                   W2_ref, b2_ref, W3_ref, b3_ref, Ws1_ref, bs1_ref,
                   Ws2_ref, bs2_ref,
                   e_ref, sid_ref, nd_ref, met_ref):
    x = conf_ref[0]
    xb = confbf_ref[0]
    n = x.shape[0]
    bf = jnp.bfloat16
    f32 = jnp.float32

    h1 = jnp.maximum(
        jnp.dot(xb, W1_ref[...], preferred_element_type=f32)
        + b1_ref[...], 0.0)
    h2 = jnp.maximum(
        jnp.dot(h1.astype(bf), W2_ref[...], preferred_element_type=f32)
        + b2_ref[...], 0.0)
    e8 = jnp.dot(h2.astype(bf), W3_ref[...], preferred_element_type=f32)
    e = e8[:, 0:1] + b3_ref[...]

    g = jnp.maximum(
        jnp.dot(xb, Ws1_ref[...], preferred_element_type=f32)
        + bs1_ref[...], 0.0)
    logits = jnp.dot(g.astype(bf), Ws2_ref[...], preferred_element_type=f32) \
        + bs2_ref[...]
    lmax = jnp.max(logits, axis=1, keepdims=True)
    cio = jax.lax.broadcasted_iota(jnp.int32, logits.shape, 1)
    sid = jnp.min(jnp.where(logits == lmax, cio, logits.shape[1]),
                  axis=1, keepdims=True)

    diff = x - native_ref[0]
    nd = jnp.sqrt(jnp.sum(diff * diff, axis=1, keepdims=True) + 1e-12)

    e_ref[0] = e
    sid_ref[0] = sid
    nd_ref[0] = nd

    nf = jnp.float32(n)
    mean_e = jnp.sum(e) / nf
    ce = e - mean_e
    se2 = jnp.sum(ce * ce)
    rugged = jnp.sqrt(se2 / (nf - 1.0))
    max_e = jnp.max(e)

    mean_nd = jnp.sum(nd) / nf
    cnd = nd - mean_nd
    snd2 = jnp.sum(cnd * cnd)
    corr = jnp.sum(cnd * ce) / jnp.sqrt(se2 * snd2)

    rio = jax.lax.broadcasted_iota(jnp.int32, nd.shape, 0)
    nd_min = jnp.min(nd)
    amin = jnp.min(jnp.where(nd == nd_min, rio, n))
    native_e = jnp.min(jnp.where(rio == amin, e, _INF))
    funnel = max_e - native_e

    lane = jax.lax.broadcasted_iota(jnp.int32, (1, 128), 1)
    met = (jnp.where(lane == 0, funnel, 0.0)
           + jnp.where(lane == 1, rugged, 0.0)
           + jnp.where(lane == 2, corr, 0.0)
           + jnp.where(lane == 3, native_e, 0.0))
    met_ref[0] = met


def _pair_body(xrow_ref, xT_ref, e_ref, sq_ref, nd_ref,
               basin_ref, depth_ref, width_ref, frus_ref, *, R, K):
    nb = pl.program_id(1)
    r0 = nb * R

    xb = xrow_ref[0]
    xT = xT_ref[0]
    e_r = e_ref[0]
    sq_r = sq_ref[0]
    nd_r = nd_ref[0]
    n = e_r.shape[1]

    dots = jnp.dot(xb, xT, preferred_element_type=jnp.float32)
    sq_b = jnp.reshape(sq_ref[0, 0, pl.ds(r0, R)], (R, 1))
    d2 = sq_b + sq_r - 2.0 * dots
    d = jnp.sqrt(jnp.maximum(d2, 0.0) + 1e-12)

    e_b = jnp.reshape(e_ref[0, 0, pl.ds(r0, R)], (R, 1))

    def _stats(thresh, neigh_min):
        is_basin = e_b <= neigh_min
        mask = d < thresh
        depth = jnp.max(jnp.where(mask, e_r, _NEG_INF), axis=1,
                        keepdims=True) - e_b
        cnt = jnp.sum(mask.astype(jnp.float32), axis=1, keepdims=True)
        wsum = jnp.sum(jnp.where(mask, d, 0.0), axis=1, keepdims=True)
        width = wsum / jnp.maximum(cnt, 1.0)
        basin_ref[0, 0] = jnp.reshape(is_basin.astype(jnp.int32), (R,))
        depth_ref[0, 0] = jnp.reshape(jnp.where(is_basin, depth, 0.0), (R,))
        width_ref[0, 0] = jnp.reshape(jnp.where(is_basin, width, 0.0), (R,))

    f32 = jnp.float32
    s1 = jnp.full((R, 128), _INF, f32)
    s2 = jnp.full((R, 128), _INF, f32)
    s3 = jnp.full((R, 128), _INF, f32)
    s4 = jnp.full((R, 128), _INF, f32)
    for g in range(n // 128):
        v = d[:, g * 128:(g + 1) * 128]
        w1 = jnp.maximum(s1, v)
        s1 = jnp.minimum(s1, v)
        w2 = jnp.maximum(s2, w1)
        s2 = jnp.minimum(s2, w1)
        w3 = jnp.maximum(s3, w2)
        s3 = jnp.minimum(s3, w2)
        s4 = jnp.minimum(s4, w3)
    cand = jnp.concatenate([s1, s2, s3, s4], axis=1)

    cm = cand
    m0 = jnp.min(cm, axis=1, keepdims=True)
    m = m0
    for _ in range(K - 1):
        cm = jnp.where(cm == m, _INF, cm)
        m = jnp.min(cm, axis=1, keepdims=True)
    cm = jnp.where(cm == m, _INF, cm)
    thresh = m
    total = jnp.sum((cm == _INF).astype(f32), axis=1, keepdims=True)

    le = d <= thresh
    cnt_le = jnp.sum(le.astype(f32), axis=1, keepdims=True)
    neigh_min = jnp.min(jnp.where(le & (d != m0), e_r, _INF),
                        axis=1, keepdims=True)
    _stats(thresh, neigh_min)

    lane_bad = jnp.any(s4 <= thresh, axis=1, keepdims=True)
    bad = (total != f32(K)) | (cnt_le != f32(K)) | lane_bad
    exact_needed = jnp.any(bad)

    @pl.when(exact_needed)
    def _():
        colio = jax.lax.broadcasted_iota(jnp.int32, (R, n), 1)
        dm2 = d
        idx0s = None
        ms = None
        for k in range(K):
            ms = jnp.min(dm2, axis=1, keepdims=True)
            idxs = jnp.min(jnp.where(dm2 == ms, colio, n), axis=1,
                           keepdims=True)
            if k == 0:
                idx0s = idxs
            dm2 = jnp.where(colio == idxs, _INF, dm2)
        nm = jnp.min(jnp.where((dm2 == _INF) & (colio != idx0s), e_r, _INF),
                     axis=1, keepdims=True)
        _stats(ms, nm)

    nd_b = jnp.reshape(nd_ref[0, 0, pl.ds(r0, R)], (R, 1))
    mlane = jnp.full((R, 128), _INF, f32)
    pe = jnp.full((R, 128), 0.0, f32)
    for g in range(n // 128):
        ndc = nd_r[:, g * 128:(g + 1) * 128]
        ec = e_r[:, g * 128:(g + 1) * 128]
        wnd = jnp.where(ndc > nd_b, ndc, _INF)
        upd = wnd < mlane
        mlane = jnp.where(upd, wnd, mlane)
        pe = jnp.where(upd, jnp.broadcast_to(ec, upd.shape), pe)
    m2 = jnp.min(mlane, axis=1, keepdims=True)
    e_succ = jnp.min(jnp.where(mlane == m2, pe, _INF),
                     axis=1, keepdims=True)
    contrib = ((e_succ > e_b) & (m2 < _INF)).astype(jnp.float32)
    part = jnp.sum(contrib)

    @pl.when(nb == 0)
    def _():
        frus_ref[0, 0] = jnp.zeros((128,), dtype=jnp.float32)

    frus_ref[0, 0] = frus_ref[0, 0] + part


def kernel(conformations, native_state, W1, b1, W2, b2, W3, b3,
           Ws1, bs1, Ws2, bs2):
    B, N, D = conformations.shape
    H = W1.shape[1]
    H2 = W2.shape[1]
    K = 10
    R = 512
    NB = N // R

    f32 = jnp.float32
    bf = jnp.bfloat16
    conf = conformations.astype(f32)
    conf_bf = conf.astype(bf)
    confT_bf = jnp.transpose(conf_bf, (0, 2, 1))

    sq = jnp.sum(conf * conf, axis=-1)
    sq_row = jnp.reshape(sq, (B, 1, N))

    C = Ws2.shape[1]
    Ws2p = jnp.zeros((H, 8), f32).at[:, :C].set(Ws2).astype(bf)
    bs2p = jnp.full((1, 8), -1e30, f32).at[0, :C].set(bs2)
    W3p = jnp.zeros((H2, 8), f32).at[:, 0:1].set(W3).astype(bf)

    b1r = jnp.reshape(b1, (1, H))
    b2r = jnp.reshape(b2, (1, H2))
    b3r = jnp.reshape(b3, (1, 1))
    bs1r = jnp.reshape(bs1, (1, H))
    nat = jnp.reshape(native_state, (B, 1, D))

    W1b = W1.astype(bf)
    W2b = W2.astype(bf)
    Ws1b = Ws1.astype(bf)

    wspec = lambda *shape: pl.BlockSpec(shape, lambda b: (0,) * len(shape))

    e_col, sid_col, nd_col, met = pl.pallas_call(
        _prologue_body,
        grid=(B,),
        in_specs=[
            pl.BlockSpec((1, N, D), lambda b: (b, 0, 0)),
            pl.BlockSpec((1, N, D), lambda b: (b, 0, 0)),
            pl.BlockSpec((1, 1, D), lambda b: (b, 0, 0)),
            wspec(D, H), wspec(1, H),
            wspec(H, H2), wspec(1, H2),
            wspec(H2, 8), wspec(1, 1),
            wspec(D, H), wspec(1, H),
            wspec(H, 8), wspec(1, 8),
        ],
        out_specs=[
            pl.BlockSpec((1, N, 1), lambda b: (b, 0, 0)),
            pl.BlockSpec((1, N, 1), lambda b: (b, 0, 0)),
            pl.BlockSpec((1, N, 1), lambda b: (b, 0, 0)),
            pl.BlockSpec((1, 1, 128), lambda b: (b, 0, 0)),
        ],
        out_shape=[
            jax.ShapeDtypeStruct((B, N, 1), f32),
            jax.ShapeDtypeStruct((B, N, 1), jnp.int32),
            jax.ShapeDtypeStruct((B, N, 1), f32),
            jax.ShapeDtypeStruct((B, 1, 128), f32),
        ],
    )(conf, conf_bf, nat, W1b, b1r, W2b, b2r, W3p, b3r, Ws1b, bs1r,
      Ws2p, bs2p)

    e_row = jnp.reshape(e_col, (B, 1, N))
    nd_row = jnp.reshape(nd_col, (B, 1, N))

    basin, depth, width, frus = pl.pallas_call(
        functools.partial(_pair_body, R=R, K=K),
        grid=(B, NB),
        in_specs=[
            pl.BlockSpec((1, R, D), lambda b, nb: (b, nb, 0)),
            pl.BlockSpec((1, D, N), lambda b, nb: (b, 0, 0)),
            pl.BlockSpec((1, 1, N), lambda b, nb: (b, 0, 0)),
            pl.BlockSpec((1, 1, N), lambda b, nb: (b, 0, 0)),
            pl.BlockSpec((1, 1, N), lambda b, nb: (b, 0, 0)),
        ],
        out_specs=[
            pl.BlockSpec((1, 1, R), lambda b, nb: (b * NB + nb, 0, 0)),
            pl.BlockSpec((1, 1, R), lambda b, nb: (b * NB + nb, 0, 0)),
            pl.BlockSpec((1, 1, R), lambda b, nb: (b * NB + nb, 0, 0)),
            pl.BlockSpec((1, 1, 128), lambda b, nb: (b, 0, 0)),
        ],
        out_shape=[
            jax.ShapeDtypeStruct((B * NB, 1, R), jnp.int32),
            jax.ShapeDtypeStruct((B * NB, 1, R), f32),
            jax.ShapeDtypeStruct((B * NB, 1, R), f32),
            jax.ShapeDtypeStruct((B, 1, 128), f32),
        ],
    )(conf_bf, confT_bf, e_row, sq_row, nd_row)

    e_out = jnp.reshape(e_col, (B, N))
    is_basin = jnp.reshape(basin, (B, N)) != 0
    depth_out = jnp.reshape(depth, (B, N))
    width_out = jnp.reshape(width, (B, N))
    sid_out = jnp.reshape(sid_col, (B, N))

    frustration = frus[:, 0, 0] / jnp.float32(N - 1)
    metrics = jnp.stack(
        [met[:, 0, 0], met[:, 0, 1], met[:, 0, 2], frustration,
         met[:, 0, 3]], axis=1)

    return (e_out, is_basin, depth_out, width_out, sid_out, metrics)
